# baseline (speedup 1.0000x reference)
import functools

import jax
import jax.numpy as jnp
from jax import lax
from jax.experimental import pallas as pl
from jax.experimental.pallas import tpu as pltpu
from jax.experimental.pallas import tpu_sc as plsc

S = 3
L = 50
B = 16384
D = 64
NC, NS = 2, 16
NW = NC * NS
COLS = B // NW
NCH = 4
CH = COLS // NCH
VG = D // 16


@functools.partial(
    pl.kernel,
    out_type=jax.ShapeDtypeStruct((S, B, D), jnp.float32),
    mesh=plsc.VectorSubcoreMesh(
        core_axis_name="c", subcore_axis_name="s", num_cores=NC, num_subcores=NS
    ),
    scratch_types=[
        [pltpu.VMEM((COLS,), jnp.int32) for _ in range(2)],
        [[pltpu.VMEM((CH, D), jnp.float32) for _ in range(NCH)] for _ in range(2)],
        pltpu.VMEM((COLS, D), jnp.float32),
        [pltpu.SemaphoreType.DMA for _ in range(2)],
        [[pltpu.SemaphoreType.DMA for _ in range(NCH)] for _ in range(2)],
    ],
    compiler_params=pltpu.CompilerParams(use_tc_tiling_on_sc=False),
)
def _sc_pool(idx_hbm, emb_hbm, out_hbm, idxbufs, rowbufs, outbuf, isems, gsems):
    wid = lax.axis_index("s") * NC + lax.axis_index("c")
    col0 = wid * COLS

    def issue_gathers(p):
        for k in range(NCH):
            pltpu.async_copy(
                emb_hbm.at[idxbufs[p].at[pl.ds(k * CH, CH)]],
                rowbufs[p][k],
                gsems[p][k],
            )

    def accum_chunk(p, k, first):
        rb = rowbufs[p][k]

        def body(i4, _):
            for u in range(4):
                i = i4 * 4 + u
                for g in range(VG):
                    v = rb[i, pl.ds(g * 16, 16)]
                    if first:
                        outbuf[k * CH + i, pl.ds(g * 16, 16)] = v
                    else:
                        plsc.addupdate(outbuf.at[k * CH + i, pl.ds(g * 16, 16)], v)
            return 0

        lax.fori_loop(0, CH // 4, body, 0)

    def make_token_body(s, first):
        def token_body(p, l):
            q = 1 - p

            @pl.when(l + 1 < L)
            def _wait_next_idx():
                pltpu.make_async_copy(
                    idx_hbm.at[s, 0, pl.ds(col0, COLS)], idxbufs[q], isems[q]
                ).wait()

            for k in range(NCH):
                pltpu.make_async_copy(
                    emb_hbm.at[pl.ds(0, CH)], rowbufs[p][k], gsems[p][k]
                ).wait()

                @pl.when(l + 1 < L)
                def _next_gather():
                    pltpu.async_copy(
                        emb_hbm.at[idxbufs[q].at[pl.ds(k * CH, CH)]],
                        rowbufs[q][k],
                        gsems[q][k],
                    )

                accum_chunk(p, k, first)

            @pl.when(l + 2 < L)
            def _next_idx():
                pltpu.async_copy(
                    idx_hbm.at[s, l + 2, pl.ds(col0, COLS)], idxbufs[p], isems[p]
                )

        return token_body

    for s in range(S):
        pltpu.sync_copy(idx_hbm.at[s, 0, pl.ds(col0, COLS)], idxbufs[0])
        pltpu.async_copy(idx_hbm.at[s, 1, pl.ds(col0, COLS)], idxbufs[1], isems[1])
        issue_gathers(0)

        make_token_body(s, True)(0, 0)
        make_token_body(s, False)(1, 1)

        def pair_add(lp, _):
            tb = make_token_body(s, False)
            tb(0, lp * 2)
            tb(1, lp * 2 + 1)
            return 0

        lax.fori_loop(1, L // 2, pair_add, 0)

        pltpu.sync_copy(outbuf, out_hbm.at[s, pl.ds(col0, COLS)])


def _mlp_body(pool_ref, w1_ref, b1_ref, w2_ref, b2_ref, out_ref):
    f32 = jnp.float32
    h = (
        jnp.dot(pool_ref[0], w1_ref[0:D, :], preferred_element_type=f32)
        + jnp.dot(pool_ref[1], w1_ref[D : 2 * D, :], preferred_element_type=f32)
        + jnp.dot(pool_ref[2], w1_ref[2 * D : 3 * D, :], preferred_element_type=f32)
    )
    h = h * (1.0 / B) + b1_ref[0]
    h = jnp.maximum(h, 0.0)
    o = jnp.dot(h, w2_ref[...], preferred_element_type=f32) + b2_ref[0]
    m = jnp.max(o, axis=1, keepdims=True)
    e = o - m
    out_ref[...] = e - jnp.log(jnp.sum(jnp.exp(e), axis=1, keepdims=True))


def _tc_mlp(pooled3, W1, b1, W2, b2):
    BLK = 2048
    grid = (B // BLK,)
    return pl.pallas_call(
        _mlp_body,
        grid=grid,
        in_specs=[
            pl.BlockSpec((S, BLK, D), lambda i: (0, i, 0)),
            pl.BlockSpec((S * D, 128), lambda i: (0, 0)),
            pl.BlockSpec((1, 128), lambda i: (0, 0)),
            pl.BlockSpec((128, 10), lambda i: (0, 0)),
            pl.BlockSpec((1, 10), lambda i: (0, 0)),
        ],
        out_specs=pl.BlockSpec((BLK, 10), lambda i: (i, 0)),
        out_shape=jax.ShapeDtypeStruct((B, 10), jnp.float32),
    )(pooled3, W1, b1, W2, b2)


def kernel(batch_inputs, emb, W1, b1, W2, b2):
    pooled3 = _sc_pool(batch_inputs, emb)
    return _tc_mlp(pooled3, W1, b1.reshape(1, -1), W2, b2.reshape(1, -1))


# speedup vs baseline: 1.1539x; 1.1539x over previous
import functools

import jax
import jax.numpy as jnp
from jax import lax
from jax.experimental import pallas as pl
from jax.experimental.pallas import tpu as pltpu
from jax.experimental.pallas import tpu_sc as plsc

S = 3
L = 50
B = 16384
D = 64
NC, NS = 2, 16
NW = NC * NS
COLS = B // NW
NCH = 4
CH = COLS // NCH
VG = D // 16


@functools.partial(
    pl.kernel,
    out_type=jax.ShapeDtypeStruct((S, B, D), jnp.float32),
    mesh=plsc.VectorSubcoreMesh(
        core_axis_name="c", subcore_axis_name="s", num_cores=NC, num_subcores=NS
    ),
    scratch_types=[
        [pltpu.VMEM((COLS,), jnp.int32) for _ in range(2)],
        [[pltpu.VMEM((CH, D), jnp.float32) for _ in range(NCH)] for _ in range(2)],
        pltpu.VMEM((COLS, D), jnp.float32),
        [pltpu.SemaphoreType.DMA for _ in range(2)],
        [[pltpu.SemaphoreType.DMA for _ in range(NCH)] for _ in range(2)],
    ],
    compiler_params=pltpu.CompilerParams(use_tc_tiling_on_sc=False),
)
def _sc_pool(idx_hbm, emb_hbm, out_hbm, idxbufs, rowbufs, outbuf, isems, gsems):
    wid = lax.axis_index("s") * NC + lax.axis_index("c")
    col0 = wid * COLS

    def issue_gathers(p):
        for k in range(NCH):
            pltpu.async_copy(
                emb_hbm.at[idxbufs[p].at[pl.ds(k * CH, CH)]],
                rowbufs[p][k],
                gsems[p][k],
            )

    def accum_chunk(p, k, first):
        rb = rowbufs[p][k]

        def body(i4, _):
            for u in range(4):
                i = i4 * 4 + u
                for g in range(VG):
                    v = rb[i, pl.ds(g * 16, 16)]
                    if first:
                        outbuf[k * CH + i, pl.ds(g * 16, 16)] = v
                    else:
                        plsc.addupdate(outbuf.at[k * CH + i, pl.ds(g * 16, 16)], v)
            return 0

        lax.fori_loop(0, CH // 4, body, 0)

    def make_token_body(s, first):
        def token_body(p, l):
            q = 1 - p
            for k in range(NCH):
                pltpu.make_async_copy(
                    emb_hbm.at[pl.ds(0, CH)], rowbufs[p][k], gsems[p][k]
                ).wait()

            @pl.when(l + 2 < L)
            def _next_idx():
                pltpu.async_copy(
                    idx_hbm.at[s, l + 2, pl.ds(col0, COLS)], idxbufs[p], isems[p]
                )

            @pl.when(l + 1 < L)
            def _next_gather():
                pltpu.make_async_copy(
                    idx_hbm.at[s, 0, pl.ds(col0, COLS)], idxbufs[q], isems[q]
                ).wait()
                issue_gathers(q)

            for k in range(NCH):
                accum_chunk(p, k, first)

        return token_body

    for s in range(S):
        pltpu.sync_copy(idx_hbm.at[s, 0, pl.ds(col0, COLS)], idxbufs[0])
        pltpu.async_copy(idx_hbm.at[s, 1, pl.ds(col0, COLS)], idxbufs[1], isems[1])
        issue_gathers(0)

        make_token_body(s, True)(0, 0)
        make_token_body(s, False)(1, 1)

        def pair_add(lp, _):
            tb = make_token_body(s, False)
            tb(0, lp * 2)
            tb(1, lp * 2 + 1)
            return 0

        lax.fori_loop(1, L // 2, pair_add, 0)

        pltpu.sync_copy(outbuf, out_hbm.at[s, pl.ds(col0, COLS)])


def _mlp_body(pool_ref, w1_ref, b1_ref, w2_ref, b2_ref, out_ref):
    f32 = jnp.float32
    h = (
        jnp.dot(pool_ref[0], w1_ref[0:D, :], preferred_element_type=f32)
        + jnp.dot(pool_ref[1], w1_ref[D : 2 * D, :], preferred_element_type=f32)
        + jnp.dot(pool_ref[2], w1_ref[2 * D : 3 * D, :], preferred_element_type=f32)
    )
    h = h * (1.0 / B) + b1_ref[0]
    h = jnp.maximum(h, 0.0)
    o = jnp.dot(h, w2_ref[...], preferred_element_type=f32) + b2_ref[0]
    m = jnp.max(o, axis=1, keepdims=True)
    e = o - m
    out_ref[...] = e - jnp.log(jnp.sum(jnp.exp(e), axis=1, keepdims=True))


def _tc_mlp(pooled3, W1, b1, W2, b2):
    BLK = 2048
    grid = (B // BLK,)
    return pl.pallas_call(
        _mlp_body,
        grid=grid,
        in_specs=[
            pl.BlockSpec((S, BLK, D), lambda i: (0, i, 0)),
            pl.BlockSpec((S * D, 128), lambda i: (0, 0)),
            pl.BlockSpec((1, 128), lambda i: (0, 0)),
            pl.BlockSpec((128, 10), lambda i: (0, 0)),
            pl.BlockSpec((1, 10), lambda i: (0, 0)),
        ],
        out_specs=pl.BlockSpec((BLK, 10), lambda i: (i, 0)),
        out_shape=jax.ShapeDtypeStruct((B, 10), jnp.float32),
    )(pooled3, W1, b1, W2, b2)


def kernel(batch_inputs, emb, W1, b1, W2, b2):
    pooled3 = _sc_pool(batch_inputs, emb)
    return _tc_mlp(pooled3, W1, b1.reshape(1, -1), W2, b2.reshape(1, -1))


# speedup vs baseline: 1.1627x; 1.0076x over previous
import functools

import jax
import jax.numpy as jnp
from jax import lax
from jax.experimental import pallas as pl
from jax.experimental.pallas import tpu as pltpu
from jax.experimental.pallas import tpu_sc as plsc

S = 3
L = 50
B = 16384
D = 64
NC, NS = 2, 16
NW = NC * NS
COLS = B // NW
NCH = 4
CH = COLS // NCH
VG = D // 16


def _make_sc_pool(nsl):
    @functools.partial(
        pl.kernel,
        out_type=jax.ShapeDtypeStruct((nsl, B, D), jnp.float32),
        mesh=plsc.VectorSubcoreMesh(
            core_axis_name="c", subcore_axis_name="s", num_cores=NC, num_subcores=NS
        ),
        scratch_types=[
            [pltpu.VMEM((COLS,), jnp.int32) for _ in range(2)],
            [[pltpu.VMEM((CH, D), jnp.float32) for _ in range(NCH)] for _ in range(2)],
            pltpu.VMEM((COLS, D), jnp.float32),
            [pltpu.SemaphoreType.DMA for _ in range(2)],
            [[pltpu.SemaphoreType.DMA for _ in range(NCH)] for _ in range(2)],
        ],
        compiler_params=pltpu.CompilerParams(use_tc_tiling_on_sc=False),
    )
    def _sc_pool(idx_hbm, emb_hbm, out_hbm, idxbufs, rowbufs, outbuf, isems, gsems):
        wid = lax.axis_index("s") * NC + lax.axis_index("c")
        col0 = wid * COLS

        def issue_gathers(p):
            for k in range(NCH):
                pltpu.async_copy(
                    emb_hbm.at[idxbufs[p].at[pl.ds(k * CH, CH)]],
                    rowbufs[p][k],
                    gsems[p][k],
                )

        def accum_chunk(p, k, first):
            rb = rowbufs[p][k]

            def body(i4, _):
                for u in range(4):
                    i = i4 * 4 + u
                    for g in range(VG):
                        v = rb[i, pl.ds(g * 16, 16)]
                        if first:
                            outbuf[k * CH + i, pl.ds(g * 16, 16)] = v
                        else:
                            plsc.addupdate(
                                outbuf.at[k * CH + i, pl.ds(g * 16, 16)], v
                            )
                return 0

            lax.fori_loop(0, CH // 4, body, 0)

        def make_token_body(s, first):
            def token_body(p, l):
                q = 1 - p
                for k in range(NCH):
                    pltpu.make_async_copy(
                        emb_hbm.at[pl.ds(0, CH)], rowbufs[p][k], gsems[p][k]
                    ).wait()

                @pl.when(l + 2 < L)
                def _next_idx():
                    pltpu.async_copy(
                        idx_hbm.at[s, l + 2, pl.ds(col0, COLS)], idxbufs[p], isems[p]
                    )

                @pl.when(l + 1 < L)
                def _next_gather():
                    pltpu.make_async_copy(
                        idx_hbm.at[s, 0, pl.ds(col0, COLS)], idxbufs[q], isems[q]
                    ).wait()
                    issue_gathers(q)

                for k in range(NCH):
                    accum_chunk(p, k, first)

            return token_body

        for s in range(nsl):
            pltpu.sync_copy(idx_hbm.at[s, 0, pl.ds(col0, COLS)], idxbufs[0])
            pltpu.async_copy(idx_hbm.at[s, 1, pl.ds(col0, COLS)], idxbufs[1], isems[1])
            issue_gathers(0)

            make_token_body(s, True)(0, 0)
            make_token_body(s, False)(1, 1)

            def pair_add(lp, _):
                tb = make_token_body(s, False)
                tb(0, lp * 2)
                tb(1, lp * 2 + 1)
                return 0

            lax.fori_loop(1, L // 2, pair_add, 0)

            pltpu.sync_copy(outbuf, out_hbm.at[s, pl.ds(col0, COLS)])

    return _sc_pool


_sc_pool_1 = _make_sc_pool(1)
_sc_pool_2 = _make_sc_pool(2)


def _mlp_body(pa_ref, pb_ref, w1_ref, b1_ref, w2_ref, b2_ref, out_ref):
    f32 = jnp.float32
    h = (
        jnp.dot(pa_ref[0], w1_ref[0:D, :], preferred_element_type=f32)
        + jnp.dot(pb_ref[0], w1_ref[D : 2 * D, :], preferred_element_type=f32)
        + jnp.dot(pb_ref[1], w1_ref[2 * D : 3 * D, :], preferred_element_type=f32)
    )
    h = h * (1.0 / B) + b1_ref[0]
    h = jnp.maximum(h, 0.0)
    o = jnp.dot(h, w2_ref[...], preferred_element_type=f32) + b2_ref[0]
    m = jnp.max(o, axis=1, keepdims=True)
    e = o - m
    out_ref[...] = e - jnp.log(jnp.sum(jnp.exp(e), axis=1, keepdims=True))


def _tc_mlp(pooled_a, pooled_b, W1, b1, W2, b2):
    BLK = 2048
    grid = (B // BLK,)
    return pl.pallas_call(
        _mlp_body,
        grid=grid,
        in_specs=[
            pl.BlockSpec((1, BLK, D), lambda i: (0, i, 0)),
            pl.BlockSpec((2, BLK, D), lambda i: (0, i, 0)),
            pl.BlockSpec((S * D, 128), lambda i: (0, 0)),
            pl.BlockSpec((1, 128), lambda i: (0, 0)),
            pl.BlockSpec((128, 10), lambda i: (0, 0)),
            pl.BlockSpec((1, 10), lambda i: (0, 0)),
        ],
        out_specs=pl.BlockSpec((BLK, 10), lambda i: (i, 0)),
        out_shape=jax.ShapeDtypeStruct((B, 10), jnp.float32),
    )(pooled_a, pooled_b, W1, b1, W2, b2)


def kernel(batch_inputs, emb, W1, b1, W2, b2):
    pooled_a = _sc_pool_1(batch_inputs[0:1], emb)
    pooled_b = _sc_pool_2(batch_inputs[1:3], emb)
    return _tc_mlp(pooled_a, pooled_b, W1, b1.reshape(1, -1), W2, b2.reshape(1, -1))


# speedup vs baseline: 1.1734x; 1.0092x over previous
import functools

import jax
import jax.numpy as jnp
from jax import lax
from jax.experimental import pallas as pl
from jax.experimental.pallas import tpu as pltpu
from jax.experimental.pallas import tpu_sc as plsc

S = 3
L = 50
B = 16384
D = 64
NC, NS = 2, 16
NW = NC * NS
COLS = B // NW
NCH = 4
CH = COLS // NCH
VG = D // 16


def _make_sc_pool(slot_lo, nsl):
    @functools.partial(
        pl.kernel,
        out_type=jax.ShapeDtypeStruct((nsl, B, D), jnp.float32),
        mesh=plsc.VectorSubcoreMesh(
            core_axis_name="c", subcore_axis_name="s", num_cores=NC, num_subcores=NS
        ),
        scratch_types=[
            [pltpu.VMEM((COLS,), jnp.int32) for _ in range(2)],
            [[pltpu.VMEM((CH, D), jnp.float32) for _ in range(NCH)] for _ in range(2)],
            pltpu.VMEM((COLS, D), jnp.float32),
            [pltpu.SemaphoreType.DMA for _ in range(2)],
            [[pltpu.SemaphoreType.DMA for _ in range(NCH)] for _ in range(2)],
        ],
        compiler_params=pltpu.CompilerParams(use_tc_tiling_on_sc=False),
    )
    def _sc_pool(idx_hbm, emb_hbm, out_hbm, idxbufs, rowbufs, outbuf, isems, gsems):
        wid = lax.axis_index("s") * NC + lax.axis_index("c")
        col0 = wid * COLS

        def issue_gathers(p):
            for k in range(NCH):
                pltpu.async_copy(
                    emb_hbm.at[idxbufs[p].at[pl.ds(k * CH, CH)]],
                    rowbufs[p][k],
                    gsems[p][k],
                )

        def accum_chunk(p, k, first):
            rb = rowbufs[p][k]

            def body(i4, _):
                for u in range(4):
                    i = i4 * 4 + u
                    for g in range(VG):
                        v = rb[i, pl.ds(g * 16, 16)]
                        if first:
                            outbuf[k * CH + i, pl.ds(g * 16, 16)] = v
                        else:
                            plsc.addupdate(
                                outbuf.at[k * CH + i, pl.ds(g * 16, 16)], v
                            )
                return 0

            lax.fori_loop(0, CH // 4, body, 0)

        def make_token_body(s, first):
            def token_body(p, l):
                q = 1 - p
                for k in range(NCH):
                    pltpu.make_async_copy(
                        emb_hbm.at[pl.ds(0, CH)], rowbufs[p][k], gsems[p][k]
                    ).wait()

                @pl.when(l + 2 < L)
                def _next_idx():
                    pltpu.async_copy(
                        idx_hbm.at[s, l + 2, pl.ds(col0, COLS)], idxbufs[p], isems[p]
                    )

                @pl.when(l + 1 < L)
                def _next_gather():
                    pltpu.make_async_copy(
                        idx_hbm.at[s, 0, pl.ds(col0, COLS)], idxbufs[q], isems[q]
                    ).wait()
                    issue_gathers(q)

                for k in range(NCH):
                    accum_chunk(p, k, first)

            return token_body

        for so in range(nsl):
            s = slot_lo + so
            pltpu.sync_copy(idx_hbm.at[s, 0, pl.ds(col0, COLS)], idxbufs[0])
            pltpu.async_copy(idx_hbm.at[s, 1, pl.ds(col0, COLS)], idxbufs[1], isems[1])
            issue_gathers(0)

            make_token_body(s, True)(0, 0)
            make_token_body(s, False)(1, 1)

            def pair_add(lp, _):
                tb = make_token_body(s, False)
                tb(0, lp * 2)
                tb(1, lp * 2 + 1)
                return 0

            lax.fori_loop(1, L // 2, pair_add, 0)

            pltpu.sync_copy(outbuf, out_hbm.at[so, pl.ds(col0, COLS)])

    return _sc_pool


_sc_pool_1 = _make_sc_pool(0, 1)
_sc_pool_2 = _make_sc_pool(1, 2)


def _mlp_body(pa_ref, pb_ref, w1_ref, b1_ref, w2_ref, b2_ref, out_ref):
    f32 = jnp.float32
    h = (
        jnp.dot(pa_ref[0], w1_ref[0:D, :], preferred_element_type=f32)
        + jnp.dot(pb_ref[0], w1_ref[D : 2 * D, :], preferred_element_type=f32)
        + jnp.dot(pb_ref[1], w1_ref[2 * D : 3 * D, :], preferred_element_type=f32)
    )
    h = h * (1.0 / B) + b1_ref[0]
    h = jnp.maximum(h, 0.0)
    o = jnp.dot(h, w2_ref[...], preferred_element_type=f32) + b2_ref[0]
    m = jnp.max(o, axis=1, keepdims=True)
    e = o - m
    out_ref[...] = e - jnp.log(jnp.sum(jnp.exp(e), axis=1, keepdims=True))


def _tc_mlp(pooled_a, pooled_b, W1, b1, W2, b2):
    BLK = 4096
    grid = (B // BLK,)
    return pl.pallas_call(
        _mlp_body,
        grid=grid,
        in_specs=[
            pl.BlockSpec((1, BLK, D), lambda i: (0, i, 0)),
            pl.BlockSpec((2, BLK, D), lambda i: (0, i, 0)),
            pl.BlockSpec((S * D, 128), lambda i: (0, 0)),
            pl.BlockSpec((1, 128), lambda i: (0, 0)),
            pl.BlockSpec((128, 10), lambda i: (0, 0)),
            pl.BlockSpec((1, 10), lambda i: (0, 0)),
        ],
        out_specs=pl.BlockSpec((BLK, 10), lambda i: (i, 0)),
        out_shape=jax.ShapeDtypeStruct((B, 10), jnp.float32),
    )(pooled_a, pooled_b, W1, b1, W2, b2)


def kernel(batch_inputs, emb, W1, b1, W2, b2):
    pooled_a = _sc_pool_1(batch_inputs, emb)
    pooled_b = _sc_pool_2(batch_inputs, emb)
    return _tc_mlp(pooled_a, pooled_b, W1, b1.reshape(1, -1), W2, b2.reshape(1, -1))
